# baseline (speedup 1.0000x reference)
import functools

import jax
import jax.numpy as jnp
from jax import lax
from jax.experimental import pallas as pl
from jax.experimental.pallas import tpu as pltpu

BATCH = 4096
SEQ = 200
D_X = 64
D_P = 32
D_O = D_X + D_P
G = 4
SG = SEQ // G
XL = G * D_X
OL = G * D_O
PL_ = G * D_P
BB = 32
RV = BB * SG
NSTEP = BATCH // BB
K = 4


def _body(x_hbm, pos_hbm, out_hbm, xbuf, obuf, ptile, pbuf, isem, osem, psem):
    pltpu.async_copy(pos_hbm, pbuf, psem).wait()
    p = pbuf[...]
    for rep in range(BB):
        ptile[rep * SG:(rep + 1) * SG, :] = p
    pt = ptile[...]

    def start_in(r, k):
        pltpu.async_copy(x_hbm.at[pl.ds(r * RV, RV)], xbuf.at[k], isem.at[k])

    def wait_in(k):
        pltpu.make_async_copy(
            x_hbm.at[pl.ds(0, RV)], xbuf.at[k], isem.at[k]).wait()

    def start_out(r, k):
        pltpu.async_copy(obuf.at[k], out_hbm.at[pl.ds(r * RV, RV)], osem.at[k])

    def wait_out(k):
        pltpu.make_async_copy(
            obuf.at[k], out_hbm.at[pl.ds(0, RV)], osem.at[k]).wait()

    for k in range(K):
        start_in(k, k)

    def step(r, _):
        k = lax.rem(r, K)
        wait_in(k)

        @pl.when(r >= K)
        def _():
            wait_out(k)

        xs = xbuf[k]
        pieces = []
        for g in range(G):
            pieces.append(xs[:, g * D_X:(g + 1) * D_X])
            pieces.append(pt[:, g * D_P:(g + 1) * D_P])
        obuf[k] = jnp.concatenate(pieces, axis=-1)
        start_out(r, k)

        @pl.when(r + K < NSTEP)
        def _():
            start_in(r + K, k)

        return 0

    lax.fori_loop(0, NSTEP, step, 0)
    for k in range(K):
        wait_out(k)


@functools.partial(jax.jit, donate_argnums=())
def kernel(x, pos_table):
    out2 = pl.pallas_call(
        _body,
        in_specs=[
            pl.BlockSpec(memory_space=pltpu.MemorySpace.HBM),
            pl.BlockSpec(memory_space=pltpu.MemorySpace.HBM),
        ],
        out_specs=pl.BlockSpec(memory_space=pltpu.MemorySpace.HBM),
        out_shape=jax.ShapeDtypeStruct((BATCH * SG, OL), jnp.float32),
        scratch_shapes=[
            pltpu.VMEM((K, RV, XL), jnp.float32),
            pltpu.VMEM((K, RV, OL), jnp.float32),
            pltpu.VMEM((RV, PL_), jnp.float32),
            pltpu.VMEM((SG, PL_), jnp.float32),
            pltpu.SemaphoreType.DMA((K,)),
            pltpu.SemaphoreType.DMA((K,)),
            pltpu.SemaphoreType.DMA,
        ],
    )(x.reshape(BATCH * SG, XL), pos_table.reshape(SG, PL_))
    return out2.reshape(BATCH, SEQ, D_O)


# speedup vs baseline: 1.0007x; 1.0007x over previous
import functools

import jax
import jax.numpy as jnp
from jax import lax
from jax.experimental import pallas as pl
from jax.experimental.pallas import tpu as pltpu

BATCH = 4096
SEQ = 200
D_X = 64
D_P = 32
D_O = D_X + D_P
G = 4
SG = SEQ // G
XL = G * D_X
OL = G * D_O
PL_ = G * D_P
BB = 32
RV = BB * SG
NSTEP = BATCH // BB
K = 4
NSPLIT = 4
RS = RV // NSPLIT


def _body(x_hbm, pos_hbm, out_hbm, xbuf, obuf, ptile, pbuf, isem, osem, psem):
    pltpu.async_copy(pos_hbm, pbuf, psem).wait()
    p = pbuf[...]
    for rep in range(BB):
        ptile[rep * SG:(rep + 1) * SG, :] = p
    pt = ptile[...]

    def start_in(r, k):
        for j in range(NSPLIT):
            pltpu.async_copy(
                x_hbm.at[pl.ds(r * RV + j * RS, RS)],
                xbuf.at[k, pl.ds(j * RS, RS)], isem.at[k, j])

    def wait_in(k):
        for j in range(NSPLIT):
            pltpu.make_async_copy(
                x_hbm.at[pl.ds(0, RS)],
                xbuf.at[k, pl.ds(j * RS, RS)], isem.at[k, j]).wait()

    def start_out(r, k):
        for j in range(NSPLIT):
            pltpu.async_copy(
                obuf.at[k, pl.ds(j * RS, RS)],
                out_hbm.at[pl.ds(r * RV + j * RS, RS)], osem.at[k, j])

    def wait_out(k):
        for j in range(NSPLIT):
            pltpu.make_async_copy(
                obuf.at[k, pl.ds(j * RS, RS)],
                out_hbm.at[pl.ds(0, RS)], osem.at[k, j]).wait()

    for k in range(K):
        start_in(k, k)

    def step(r, _):
        k = lax.rem(r, K)
        wait_in(k)

        @pl.when(r >= K)
        def _():
            wait_out(k)

        xs = xbuf[k]
        pieces = []
        for g in range(G):
            pieces.append(xs[:, g * D_X:(g + 1) * D_X])
            pieces.append(pt[:, g * D_P:(g + 1) * D_P])
        obuf[k] = jnp.concatenate(pieces, axis=-1)
        start_out(r, k)

        @pl.when(r + K < NSTEP)
        def _():
            start_in(r + K, k)

        return 0

    lax.fori_loop(0, NSTEP, step, 0)
    for k in range(K):
        wait_out(k)


@functools.partial(jax.jit, donate_argnums=())
def kernel(x, pos_table):
    out2 = pl.pallas_call(
        _body,
        in_specs=[
            pl.BlockSpec(memory_space=pltpu.MemorySpace.HBM),
            pl.BlockSpec(memory_space=pltpu.MemorySpace.HBM),
        ],
        out_specs=pl.BlockSpec(memory_space=pltpu.MemorySpace.HBM),
        out_shape=jax.ShapeDtypeStruct((BATCH * SG, OL), jnp.float32),
        scratch_shapes=[
            pltpu.VMEM((K, RV, XL), jnp.float32),
            pltpu.VMEM((K, RV, OL), jnp.float32),
            pltpu.VMEM((RV, PL_), jnp.float32),
            pltpu.VMEM((SG, PL_), jnp.float32),
            pltpu.SemaphoreType.DMA((K, NSPLIT)),
            pltpu.SemaphoreType.DMA((K, NSPLIT)),
            pltpu.SemaphoreType.DMA,
        ],
    )(x.reshape(BATCH * SG, XL), pos_table.reshape(SG, PL_))
    return out2.reshape(BATCH, SEQ, D_O)


# speedup vs baseline: 1.9141x; 1.9128x over previous
import functools

import jax
import jax.numpy as jnp
from jax.experimental import pallas as pl

BATCH = 4096
SEQ = 200
D_X = 64
D_P = 32
D_O = D_X + D_P
G = 4
SG = SEQ // G
BB = 128


def _body(x_ref, pos_ref, out_ref):
    x = x_ref[...]
    p = jnp.broadcast_to(pos_ref[...][None, :, :], (BB, SG, G * D_P))
    pieces = []
    for g in range(G):
        pieces.append(x[:, :, g * D_X:(g + 1) * D_X])
        pieces.append(p[:, :, g * D_P:(g + 1) * D_P])
    out_ref[...] = jnp.concatenate(pieces, axis=-1)


@functools.partial(jax.jit, donate_argnums=())
def kernel(x, pos_table):
    out3 = pl.pallas_call(
        _body,
        grid=(BATCH // BB,),
        in_specs=[
            pl.BlockSpec((BB, SG, G * D_X), lambda i: (i, 0, 0)),
            pl.BlockSpec((SG, G * D_P), lambda i: (0, 0)),
        ],
        out_specs=pl.BlockSpec((BB, SG, G * D_O), lambda i: (i, 0, 0)),
        out_shape=jax.ShapeDtypeStruct((BATCH, SG, G * D_O), jnp.float32),
    )(x.reshape(BATCH, SG, G * D_X), pos_table.reshape(SG, G * D_P))
    return out3.reshape(BATCH, SEQ, D_O)
